# baseline (speedup 1.0000x reference)
import jax
import jax.numpy as jnp
from jax import lax
from jax.experimental import pallas as pl
from jax.experimental.pallas import tpu as pltpu
from jax.experimental.pallas import tpu_sc as plsc

N_DIM = 64
PADDING = 20
PACK = 4
PACKW = PACK * PADDING
BATCH = 16384
LANES = 16
BPW = BATCH // 32
CHUNK = 128
NCH = BPW // CHUNK


def _fasttext_body(x_hbm, wts_hbm, ez_hbm, ev_hbm, y_hbm,
                   xs_v, idx1_v, idxb_v, idx2_v, wts_v, cols_v, u_v, v_v, y_v,
                   sem_x, sems_w, sems_v, sems_z):
    wid = lax.axis_index("s") * 2 + lax.axis_index("c")
    base = wid * BPW
    iota = lax.iota(jnp.int32, LANES)
    zeros16 = jnp.zeros((LANES,), jnp.float32)
    bpc = CHUNK // LANES

    scope = jax.named_scope
    with scope("xsplit"):
        pltpu.async_copy(x_hbm.at[pl.ds(base, BPW)], xs_v, sem_x).wait()

    def split_x(i, c):
        ch = i // bpc
        off = (i % bpc) * LANES
        rows = i * LANES + iota
        x1v = plsc.load_gather(xs_v, [rows, jnp.zeros((LANES,), jnp.int32)])
        x2v = plsc.load_gather(xs_v, [rows, jnp.ones((LANES,), jnp.int32)])
        idx1_v[ch, pl.ds(off, LANES)] = x1v
        idxb_v[ch, pl.ds(off, LANES)] = x1v >> 2
        idx2_v[ch, pl.ds(off, LANES)] = x2v
        return c

    lax.fori_loop(0, BPW // LANES, split_x, 0)

    def fire_rows(c, _):
        pltpu.async_copy(wts_hbm.at[idxb_v.at[c]],
                         wts_v.at[pl.ds(c * CHUNK, CHUNK)], sems_w.at[c])
        pltpu.async_copy(ev_hbm.at[idx2_v.at[c]],
                         v_v.at[pl.ds(c * CHUNK, CHUNK)], sems_v.at[c])
        return _

    with scope("fire_rows"):
        lax.fori_loop(0, NCH, fire_rows, 0)

    def zero_row(i, c):
        for k in range(N_DIM // LANES):
            u_v[i, pl.ds(k * LANES, LANES)] = zeros16
        return c

    with scope("zero"):
        lax.fori_loop(0, BPW, zero_row, 0)

    def stage_chunk(c, _):
        pltpu.make_async_copy(wts_hbm.at[idxb_v.at[c]],
                              wts_v.at[pl.ds(c * CHUNK, CHUNK)],
                              sems_w.at[c]).wait()

        def build_cols(j, carry):
            i = c * bpc + j
            rows = i * LANES + iota
            off = j * LANES
            x1v = idx1_v[c, pl.ds(off, LANES)]
            colbase = (x1v & (PACK - 1)) * PADDING

            def one_col(p, cc):
                vals = plsc.load_gather(wts_v, [rows, colbase + p])
                cols_v[p * NCH + c, pl.ds(off, LANES)] = vals
                return cc

            lax.fori_loop(0, PADDING, one_col, 0)
            return carry

        lax.fori_loop(0, bpc, build_cols, 0)

        def fire_bag(p, cc):
            pltpu.async_copy(ez_hbm.at[cols_v.at[p * NCH + c]],
                             u_v.at[pl.ds(c * CHUNK, CHUNK)],
                             sems_z.at[c], add=True)
            return cc

        lax.fori_loop(0, PADDING, fire_bag, 0)
        return _

    with scope("stage_chunks"):
        lax.fori_loop(0, NCH, stage_chunk, 0)

    last_lane = iota == (LANES - 1)

    def dot_row(b, c):
        acc = zeros16
        for k in range(N_DIM // LANES):
            sl = pl.ds(k * LANES, LANES)
            acc = acc + u_v[b, sl] * v_v[b, sl]
        s = plsc.cumsum(acc)
        plsc.store_scatter(y_v, [jnp.full((LANES,), 0, jnp.int32) + b], s,
                           mask=last_lane)
        return c

    def finish_chunk(c, _):
        def drain(p, cc):
            pltpu.make_async_copy(ez_hbm.at[cols_v.at[p * NCH + c]],
                                  u_v.at[pl.ds(c * CHUNK, CHUNK)],
                                  sems_z.at[c]).wait()
            return cc

        lax.fori_loop(0, PADDING, drain, 0)
        pltpu.make_async_copy(ev_hbm.at[idx2_v.at[c]],
                              v_v.at[pl.ds(c * CHUNK, CHUNK)],
                              sems_v.at[c]).wait()
        lax.fori_loop(c * CHUNK, (c + 1) * CHUNK, dot_row, 0)
        return _

    with scope("drain_dot"):
        lax.fori_loop(0, NCH, finish_chunk, 0)

    pltpu.sync_copy(y_v, y_hbm.at[pl.ds(base, BPW)])


@jax.jit
def kernel(x, word_to_sub, embedding_z, embedding_v):
    wts_packed = word_to_sub.reshape(word_to_sub.shape[0] // PACK, PACKW)
    mesh = plsc.VectorSubcoreMesh(core_axis_name="c", subcore_axis_name="s",
                                  num_cores=2, num_subcores=16)
    kfn = pl.kernel(
        _fasttext_body,
        out_type=jax.ShapeDtypeStruct((BATCH,), jnp.float32),
        mesh=mesh,
        scratch_types=[
            pltpu.VMEM((BPW, 2), jnp.int32),
            pltpu.VMEM((NCH, CHUNK), jnp.int32),
            pltpu.VMEM((NCH, CHUNK), jnp.int32),
            pltpu.VMEM((NCH, CHUNK), jnp.int32),
            pltpu.VMEM((BPW, PACKW), jnp.int32),
            pltpu.VMEM((PADDING * NCH, CHUNK), jnp.int32),
            pltpu.VMEM((BPW, N_DIM), jnp.float32),
            pltpu.VMEM((BPW, N_DIM), jnp.float32),
            pltpu.VMEM((BPW,), jnp.float32),
            pltpu.SemaphoreType.DMA,
            pltpu.SemaphoreType.DMA((NCH,)),
            pltpu.SemaphoreType.DMA((NCH,)),
            pltpu.SemaphoreType.DMA((NCH,)),
        ],
        compiler_params=pltpu.CompilerParams(needs_layout_passes=False,
                                             use_tc_tiling_on_sc=False),
    )
    return kfn(x, wts_packed, embedding_z, embedding_v)
